# Initial kernel scaffold; baseline (speedup 1.0000x reference)
import jax
import jax.numpy as jnp
from jax.experimental import pallas as pl


def kernel(nodefeature, adjm, params):
    raise NotImplementedError("write your pallas kernel here")


# speedup vs baseline: 7.9100x; 7.9100x over previous
import math

import jax
import jax.numpy as jnp
from jax.experimental import pallas as pl

N = 512
F_IN = 64
H = 8
DH = 16
D = 128
L = 2
TOPK = 16


def _matT(x, w):
    return jax.lax.dot_general(
        x, w, (((x.ndim - 1,), (w.ndim - 1,)), ((), ())),
        preferred_element_type=jnp.float32)


def _layernorm(x, w, b, eps=1e-5):
    mu = jnp.mean(x, axis=-1, keepdims=True)
    var = jnp.mean((x - mu) ** 2, axis=-1, keepdims=True)
    return w * (x - mu) / jnp.sqrt(var + eps) + b


def _topk_mask(s):
    iota = jax.lax.broadcasted_iota(jnp.int32, s.shape, 1)

    def body(_, w):
        m = jnp.max(w, axis=1, keepdims=True)
        first = jnp.min(jnp.where(w == m, iota, N), axis=1, keepdims=True)
        return jnp.where(iota == first, -jnp.inf, w)

    w = jax.lax.fori_loop(0, TOPK, body, s)
    return w == -jnp.inf


def _mynet_body(nf_ref, adjm_ref, emb_W_ref, emb_b_ref, edge_w_ref,
                edge_b_ref, Wq_ref, bq_ref, Wk_ref, bk_ref, Wv_ref, bv_ref,
                We_ref, be_ref, Wo_ref, bo_ref, ln1w_ref, ln1b_ref,
                lstm_Wih_ref, lstm_bih_ref, lstm_bhh_ref, gn_g_ref, gn_b_ref,
                ffn_W1_ref, ffn_b1_ref, ffn_W2_ref, ffn_b2_ref, ln2w_ref,
                ln2b_ref, out_W_ref, out_b_ref, o_ref):
    f32 = jnp.float32
    h = _matT(nf_ref[...], emb_W_ref[...]) + emb_b_ref[...]
    h0 = h
    binm = (adjm_ref[...] > 0).astype(f32)
    be_all = be_ref[...]
    inv_sqrt_dh = 1.0 / math.sqrt(DH)

    for l in range(L):
        residual = h
        q = _matT(h, Wq_ref[l]) + bq_ref[l:l + 1]
        k = _matT(h, Wk_ref[l]) + bk_ref[l:l + 1]
        v = _matT(h, Wv_ref[l]) + bv_ref[l:l + 1]
        We_l = We_ref[l]
        ww = _matT(We_l, edge_w_ref[...])
        bb = _matT(We_l, edge_b_ref[...])
        Wo_l = Wo_ref[l]

        acc = jnp.zeros((N, D), f32)
        for hd in range(H):
            sl = slice(hd * DH, (hd + 1) * DH)
            q_h = q[:, sl]
            k_h = k[:, sl]
            v_h = v[:, sl]
            s_h = _matT(q_h, k_h) * inv_sqrt_dh
            s_h = s_h + (bb[hd, 0] + be_all[l, hd]) + binm * ww[hd, 0]
            selm = _topk_mask(s_h)
            sm = jnp.where(selm, s_h, -jnp.inf)
            sm = sm - jnp.max(sm, axis=1, keepdims=True)
            p = jnp.exp(sm)
            attn = p / jnp.sum(p, axis=1, keepdims=True)
            av = jax.lax.dot_general(
                attn, v_h, (((1,), (0,)), ((), ())),
                preferred_element_type=f32)
            acc = acc + _matT(av, Wo_l[:, sl])

        h_attn = acc + bo_ref[l:l + 1]
        h = _layernorm(residual + h_attn, ln1w_ref[l:l + 1], ln1b_ref[l:l + 1])

        gates = (_matT(h, lstm_Wih_ref[l]) + lstm_bih_ref[l:l + 1]
                 + lstm_bhh_ref[l:l + 1])
        i_g = jax.nn.sigmoid(gates[:, 0:D])
        g_g = jnp.tanh(gates[:, 2 * D:3 * D])
        o_g = jax.nn.sigmoid(gates[:, 3 * D:4 * D])
        h = h + o_g * jnp.tanh(i_g * g_g)

        mu = jnp.mean(h, axis=0, keepdims=True)
        var = jnp.mean((h - mu) ** 2, axis=0, keepdims=True)
        h = gn_g_ref[l:l + 1] * (h - mu) / jnp.sqrt(var + 1e-5) \
            + gn_b_ref[l:l + 1]

        a = jnp.maximum(_matT(h, ffn_W1_ref[l]) + ffn_b1_ref[l:l + 1], 0.0)
        f = _matT(a, ffn_W2_ref[l]) + ffn_b2_ref[l:l + 1]
        h = _layernorm(h + f, ln2w_ref[l:l + 1], ln2b_ref[l:l + 1])
        h = h + h0

    o_ref[...] = _matT(h, out_W_ref[...]) + out_b_ref[...]


def kernel(nodefeature, adjm, params):
    p = params
    args = (
        nodefeature, adjm,
        p['emb_W'], p['emb_b'].reshape(1, D),
        p['edge_W'].T, p['edge_b'].reshape(1, D),
        p['Wq'], p['bq'], p['Wk'], p['bk'], p['Wv'], p['bv'],
        p['We'], p['be'], p['Wo'], p['bo'],
        p['ln1_w'], p['ln1_b'],
        p['lstm_Wih'], p['lstm_bih'], p['lstm_bhh'],
        p['gn_g'], p['gn_b'],
        p['ffn_W1'], p['ffn_b1'], p['ffn_W2'], p['ffn_b2'],
        p['ln2_w'], p['ln2_b'],
        p['out_W'], p['out_b'].reshape(1, D),
    )
    return pl.pallas_call(
        _mynet_body,
        out_shape=jax.ShapeDtypeStruct((N, D), jnp.float32),
    )(*args)


# speedup vs baseline: 9.6789x; 1.2236x over previous
import math

import jax
import jax.numpy as jnp
from jax.experimental import pallas as pl
from jax.experimental.pallas import tpu as pltpu

N = 512
F_IN = 64
H = 8
DH = 16
D = 128
L = 2
TOPK = 16


def _matT(x, w):
    return jax.lax.dot_general(
        x, w, (((x.ndim - 1,), (w.ndim - 1,)), ((), ())),
        preferred_element_type=jnp.float32)


def _layernorm(x, w, b, eps=1e-5):
    mu = jnp.mean(x, axis=-1, keepdims=True)
    var = jnp.mean((x - mu) ** 2, axis=-1, keepdims=True)
    return w * (x - mu) / jnp.sqrt(var + eps) + b


def _topk_mask(s):
    f32 = jnp.float32
    rows = s.shape[0]

    def scoped(cnt_ref, t_ref, g_ref):
        cnt_ref[...] = jnp.zeros((rows, 1), f32)
        t_ref[...] = jnp.zeros((rows, 1), f32)
        g_ref[...] = jnp.zeros((rows, 1), f32)

        def body(_, w):
            m = jnp.max(w, axis=1, keepdims=True)
            hit = w == m
            nh = jnp.sum(hit.astype(f32), axis=1, keepdims=True)
            cnt = cnt_ref[...]
            live = cnt < float(TOPK)
            t_ref[...] = jnp.where(live, m, t_ref[...])
            g_ref[...] = jnp.where(live, cnt, g_ref[...])
            cnt_ref[...] = cnt + nh
            return jnp.where(hit, -jnp.inf, w)

        jax.lax.fori_loop(0, TOPK, body, s)
        return t_ref[...], g_ref[...]

    t, g = pl.run_scoped(scoped,
                         pltpu.VMEM((rows, 1), f32),
                         pltpu.VMEM((rows, 1), f32),
                         pltpu.VMEM((rows, 1), f32))
    eq = s == t
    tri = (jax.lax.broadcasted_iota(jnp.int32, (N, N), 0)
           <= jax.lax.broadcasted_iota(jnp.int32, (N, N), 1)).astype(f32)
    cum = jax.lax.dot_general(eq.astype(f32), tri, (((1,), (0,)), ((), ())),
                              preferred_element_type=f32)
    return (s > t) | (eq & (cum <= (float(TOPK) - g)))


def _mynet_body(nf_ref, adjm_ref, emb_W_ref, emb_b_ref, edge_w_ref,
                edge_b_ref, Wq_ref, bq_ref, Wk_ref, bk_ref, Wv_ref, bv_ref,
                We_ref, be_ref, Wo_ref, bo_ref, ln1w_ref, ln1b_ref,
                lstm_Wih_ref, lstm_bih_ref, lstm_bhh_ref, gn_g_ref, gn_b_ref,
                ffn_W1_ref, ffn_b1_ref, ffn_W2_ref, ffn_b2_ref, ln2w_ref,
                ln2b_ref, out_W_ref, out_b_ref, o_ref):
    f32 = jnp.float32
    h = _matT(nf_ref[...], emb_W_ref[...]) + emb_b_ref[...]
    h0 = h
    binm = (adjm_ref[...] > 0).astype(f32)
    be_all = be_ref[...]
    inv_sqrt_dh = 1.0 / math.sqrt(DH)

    for l in range(L):
        residual = h
        q = _matT(h, Wq_ref[l]) + bq_ref[l:l + 1]
        k = _matT(h, Wk_ref[l]) + bk_ref[l:l + 1]
        v = _matT(h, Wv_ref[l]) + bv_ref[l:l + 1]
        We_l = We_ref[l]
        ww = _matT(We_l, edge_w_ref[...])
        bb = _matT(We_l, edge_b_ref[...])
        Wo_l = Wo_ref[l]

        s_list = []
        for hd in range(H):
            sl = slice(hd * DH, (hd + 1) * DH)
            s_h = _matT(q[:, sl], k[:, sl]) * inv_sqrt_dh
            s_list.append(s_h + (bb[hd, 0] + be_all[l, hd])
                          + binm * ww[hd, 0])
        s_all = jnp.concatenate(s_list, axis=0)
        m1 = jnp.max(s_all, axis=1, keepdims=True)
        selm = _topk_mask(s_all)
        p = jnp.where(selm, jnp.exp(s_all - m1), 0.0)
        attn = p / jnp.sum(p, axis=1, keepdims=True)

        acc = jnp.zeros((N, D), f32)
        for hd in range(H):
            sl = slice(hd * DH, (hd + 1) * DH)
            av = jax.lax.dot_general(
                attn[hd * N:(hd + 1) * N, :], v[:, sl],
                (((1,), (0,)), ((), ())),
                preferred_element_type=f32)
            acc = acc + _matT(av, Wo_l[:, sl])

        h_attn = acc + bo_ref[l:l + 1]
        h = _layernorm(residual + h_attn, ln1w_ref[l:l + 1], ln1b_ref[l:l + 1])

        gates = (_matT(h, lstm_Wih_ref[l]) + lstm_bih_ref[l:l + 1]
                 + lstm_bhh_ref[l:l + 1])
        i_g = jax.nn.sigmoid(gates[:, 0:D])
        g_g = jnp.tanh(gates[:, 2 * D:3 * D])
        o_g = jax.nn.sigmoid(gates[:, 3 * D:4 * D])
        h = h + o_g * jnp.tanh(i_g * g_g)

        mu = jnp.mean(h, axis=0, keepdims=True)
        var = jnp.mean((h - mu) ** 2, axis=0, keepdims=True)
        h = gn_g_ref[l:l + 1] * (h - mu) / jnp.sqrt(var + 1e-5) \
            + gn_b_ref[l:l + 1]

        a = jnp.maximum(_matT(h, ffn_W1_ref[l]) + ffn_b1_ref[l:l + 1], 0.0)
        f = _matT(a, ffn_W2_ref[l]) + ffn_b2_ref[l:l + 1]
        h = _layernorm(h + f, ln2w_ref[l:l + 1], ln2b_ref[l:l + 1])
        h = h + h0

    o_ref[...] = _matT(h, out_W_ref[...]) + out_b_ref[...]


def kernel(nodefeature, adjm, params):
    p = params
    args = (
        nodefeature, adjm,
        p['emb_W'], p['emb_b'].reshape(1, D),
        p['edge_W'].T, p['edge_b'].reshape(1, D),
        p['Wq'], p['bq'], p['Wk'], p['bk'], p['Wv'], p['bv'],
        p['We'], p['be'], p['Wo'], p['bo'],
        p['ln1_w'], p['ln1_b'],
        p['lstm_Wih'], p['lstm_bih'], p['lstm_bhh'],
        p['gn_g'], p['gn_b'],
        p['ffn_W1'], p['ffn_b1'], p['ffn_W2'], p['ffn_b2'],
        p['ln2_w'], p['ln2_b'],
        p['out_W'], p['out_b'].reshape(1, D),
    )
    return pl.pallas_call(
        _mynet_body,
        out_shape=jax.ShapeDtypeStruct((N, D), jnp.float32),
    )(*args)


# speedup vs baseline: 9.9082x; 1.0237x over previous
import math

import jax
import jax.numpy as jnp
from jax.experimental import pallas as pl
from jax.experimental.pallas import tpu as pltpu

N = 512
F_IN = 64
H = 8
DH = 16
D = 128
L = 2
TOPK = 16
CH = 128


def _matT(x, w):
    return jax.lax.dot_general(
        x, w, (((x.ndim - 1,), (w.ndim - 1,)), ((), ())),
        preferred_element_type=jnp.float32)


def _layernorm(x, w, b, eps=1e-5):
    mu = jnp.mean(x, axis=-1, keepdims=True)
    var = jnp.mean((x - mu) ** 2, axis=-1, keepdims=True)
    return w * (x - mu) / jnp.sqrt(var + eps) + b


def _topk_stats(s):
    f32 = jnp.float32
    rows = s.shape[0]

    def scoped(cnt_ref, t_ref, g_ref):
        cnt_ref[...] = jnp.zeros((rows, 1), f32)
        t_ref[...] = jnp.zeros((rows, 1), f32)
        g_ref[...] = jnp.zeros((rows, 1), f32)

        def body(_, w):
            m = jnp.max(w, axis=1, keepdims=True)
            hit = w == m
            nh = jnp.sum(hit.astype(f32), axis=1, keepdims=True)
            cnt = cnt_ref[...]
            live = cnt < float(TOPK)
            t_ref[...] = jnp.where(live, m, t_ref[...])
            g_ref[...] = jnp.where(live, cnt, g_ref[...])
            cnt_ref[...] = cnt + nh
            return jnp.where(hit, -jnp.inf, w)

        jax.lax.fori_loop(0, TOPK, body, s)
        return t_ref[...], g_ref[...]

    t, g = pl.run_scoped(scoped,
                         pltpu.VMEM((rows, 1), f32),
                         pltpu.VMEM((rows, 1), f32),
                         pltpu.VMEM((rows, 1), f32))
    return t, g


def _mynet_body(nf_ref, adjm_ref, emb_W_ref, emb_b_ref, edge_w_ref,
                edge_b_ref, Wq_ref, bq_ref, Wk_ref, bk_ref, Wv_ref, bv_ref,
                We_ref, be_ref, Wo_ref, bo_ref, ln1w_ref, ln1b_ref,
                lstm_Wih_ref, lstm_bih_ref, lstm_bhh_ref, gn_g_ref, gn_b_ref,
                ffn_W1_ref, ffn_b1_ref, ffn_W2_ref, ffn_b2_ref, ln2w_ref,
                ln2b_ref, out_W_ref, out_b_ref, o_ref):
    f32 = jnp.float32
    h = _matT(nf_ref[...], emb_W_ref[...]) + emb_b_ref[...]
    h0 = h
    binm = (adjm_ref[...] > 0).astype(f32)
    be_all = be_ref[...]
    inv_sqrt_dh = 1.0 / math.sqrt(DH)

    for l in range(L):
        residual = h
        q = _matT(h, Wq_ref[l]) + bq_ref[l:l + 1]
        k = _matT(h, Wk_ref[l]) + bk_ref[l:l + 1]
        v = _matT(h, Wv_ref[l]) + bv_ref[l:l + 1]
        We_l = We_ref[l]
        ww = _matT(We_l, edge_w_ref[...])
        bb = _matT(We_l, edge_b_ref[...])
        Wo_l = Wo_ref[l]

        s_list = []
        for hd in range(H):
            sl = slice(hd * DH, (hd + 1) * DH)
            s_h = _matT(q[:, sl], k[:, sl]) * inv_sqrt_dh
            s_list.append(s_h + (bb[hd, 0] + be_all[l, hd])
                          + binm * ww[hd, 0])
        s_all = jnp.concatenate(s_list, axis=0)
        m1 = jnp.max(s_all, axis=1, keepdims=True)
        t, g = _topk_stats(s_all)
        r = float(TOPK) - g

        tri = (jax.lax.broadcasted_iota(jnp.int32, (CH, CH), 0)
               <= jax.lax.broadcasted_iota(jnp.int32, (CH, CH), 1)
               ).astype(f32)
        p_chunks = []
        off = jnp.zeros((H * N, 1), f32)
        for c in range(N // CH):
            sc = slice(c * CH, (c + 1) * CH)
            s_c = s_all[:, sc]
            eq_c = (s_c == t)
            cum_c = jax.lax.dot_general(
                eq_c.astype(f32), tri, (((1,), (0,)), ((), ())),
                preferred_element_type=f32) + off
            off = cum_c[:, CH - 1:CH]
            sel_c = (s_c > t) | (eq_c & (cum_c <= r))
            p_chunks.append(jnp.where(sel_c, jnp.exp(s_c - m1), 0.0))

        z = p_chunks[0].sum(axis=1, keepdims=True)
        for c in range(1, N // CH):
            z = z + p_chunks[c].sum(axis=1, keepdims=True)

        acc = jnp.zeros((N, D), f32)
        for hd in range(H):
            sl = slice(hd * DH, (hd + 1) * DH)
            rows = slice(hd * N, (hd + 1) * N)
            av = jnp.zeros((N, DH), f32)
            for c in range(N // CH):
                av = av + jax.lax.dot_general(
                    p_chunks[c][rows, :], v[c * CH:(c + 1) * CH, sl],
                    (((1,), (0,)), ((), ())),
                    preferred_element_type=f32)
            acc = acc + _matT(av / z[rows, :], Wo_l[:, sl])

        h_attn = acc + bo_ref[l:l + 1]
        h = _layernorm(residual + h_attn, ln1w_ref[l:l + 1], ln1b_ref[l:l + 1])

        gates = (_matT(h, lstm_Wih_ref[l]) + lstm_bih_ref[l:l + 1]
                 + lstm_bhh_ref[l:l + 1])
        i_g = jax.nn.sigmoid(gates[:, 0:D])
        g_g = jnp.tanh(gates[:, 2 * D:3 * D])
        o_g = jax.nn.sigmoid(gates[:, 3 * D:4 * D])
        h = h + o_g * jnp.tanh(i_g * g_g)

        mu = jnp.mean(h, axis=0, keepdims=True)
        var = jnp.mean((h - mu) ** 2, axis=0, keepdims=True)
        h = gn_g_ref[l:l + 1] * (h - mu) / jnp.sqrt(var + 1e-5) \
            + gn_b_ref[l:l + 1]

        a = jnp.maximum(_matT(h, ffn_W1_ref[l]) + ffn_b1_ref[l:l + 1], 0.0)
        f = _matT(a, ffn_W2_ref[l]) + ffn_b2_ref[l:l + 1]
        h = _layernorm(h + f, ln2w_ref[l:l + 1], ln2b_ref[l:l + 1])
        h = h + h0

    o_ref[...] = _matT(h, out_W_ref[...]) + out_b_ref[...]


def kernel(nodefeature, adjm, params):
    p = params
    args = (
        nodefeature, adjm,
        p['emb_W'], p['emb_b'].reshape(1, D),
        p['edge_W'].T, p['edge_b'].reshape(1, D),
        p['Wq'], p['bq'], p['Wk'], p['bk'], p['Wv'], p['bv'],
        p['We'], p['be'], p['Wo'], p['bo'],
        p['ln1_w'], p['ln1_b'],
        p['lstm_Wih'], p['lstm_bih'], p['lstm_bhh'],
        p['gn_g'], p['gn_b'],
        p['ffn_W1'], p['ffn_b1'], p['ffn_W2'], p['ffn_b2'],
        p['ln2_w'], p['ln2_b'],
        p['out_W'], p['out_b'].reshape(1, D),
    )
    return pl.pallas_call(
        _mynet_body,
        out_shape=jax.ShapeDtypeStruct((N, D), jnp.float32),
    )(*args)


# speedup vs baseline: 16.6882x; 1.6843x over previous
import math

import jax
import jax.numpy as jnp
from jax.experimental import pallas as pl
from jax.experimental.pallas import tpu as pltpu

N = 512
F_IN = 64
H = 8
DH = 16
D = 128
L = 2
TOPK = 16
CH = 128


def _matT(x, w):
    return jax.lax.dot_general(
        x, w, (((x.ndim - 1,), (w.ndim - 1,)), ((), ())),
        preferred_element_type=jnp.float32)


def _layernorm(x, w, b, eps=1e-5):
    mu = jnp.mean(x, axis=-1, keepdims=True)
    var = jnp.mean((x - mu) ** 2, axis=-1, keepdims=True)
    return w * (x - mu) / jnp.sqrt(var + eps) + b


def _topk_stats(s):
    f32 = jnp.float32
    rows = s.shape[0]

    def scoped(cnt_ref, t_ref, g_ref):
        cnt_ref[...] = jnp.full((rows, 1), jnp.inf, f32)

        def chain(_, carry):
            cnt_ref[...] = jnp.max(
                jnp.where(s < cnt_ref[...], s, -jnp.inf),
                axis=1, keepdims=True)
            return carry

        jax.lax.fori_loop(0, TOPK, chain, 0)
        m = cnt_ref[...]
        c = jnp.sum((s >= m).astype(f32), axis=1, keepdims=True)
        t_ref[...] = m
        g_ref[...] = c - 1.0
        bad = jnp.max(jnp.abs(c - float(TOPK)))

        @pl.when(bad > 0.0)
        def _():
            cnt_ref[...] = jnp.zeros((rows, 1), f32)
            t_ref[...] = jnp.zeros((rows, 1), f32)
            g_ref[...] = jnp.zeros((rows, 1), f32)

            def body(_, w):
                mm = jnp.max(w, axis=1, keepdims=True)
                hit = w == mm
                nh = jnp.sum(hit.astype(f32), axis=1, keepdims=True)
                cnt = cnt_ref[...]
                live = cnt < float(TOPK)
                t_ref[...] = jnp.where(live, mm, t_ref[...])
                g_ref[...] = jnp.where(live, cnt, g_ref[...])
                cnt_ref[...] = cnt + nh
                return jnp.where(hit, -jnp.inf, w)

            jax.lax.fori_loop(0, TOPK, body, s)

        return t_ref[...], g_ref[...]

    return pl.run_scoped(scoped,
                         pltpu.VMEM((rows, 1), f32),
                         pltpu.VMEM((rows, 1), f32),
                         pltpu.VMEM((rows, 1), f32))


def _mynet_body(nf_ref, adjm_ref, emb_W_ref, emb_b_ref, edge_w_ref,
                edge_b_ref, Wq_ref, bq_ref, Wk_ref, bk_ref, Wv_ref, bv_ref,
                We_ref, be_ref, Wo_ref, bo_ref, ln1w_ref, ln1b_ref,
                lstm_Wih_ref, lstm_bih_ref, lstm_bhh_ref, gn_g_ref, gn_b_ref,
                ffn_W1_ref, ffn_b1_ref, ffn_W2_ref, ffn_b2_ref, ln2w_ref,
                ln2b_ref, out_W_ref, out_b_ref, o_ref):
    f32 = jnp.float32
    h = _matT(nf_ref[...], emb_W_ref[...]) + emb_b_ref[...]
    h0 = h
    binm = (adjm_ref[...] > 0).astype(f32)
    be_all = be_ref[...]
    inv_sqrt_dh = 1.0 / math.sqrt(DH)

    for l in range(L):
        residual = h
        q = _matT(h, Wq_ref[l]) + bq_ref[l:l + 1]
        k = _matT(h, Wk_ref[l]) + bk_ref[l:l + 1]
        v = _matT(h, Wv_ref[l]) + bv_ref[l:l + 1]
        We_l = We_ref[l]
        ww = _matT(We_l, edge_w_ref[...])
        bb = _matT(We_l, edge_b_ref[...])
        Wo_l = Wo_ref[l]

        s_list = []
        for hd in range(H):
            sl = slice(hd * DH, (hd + 1) * DH)
            s_h = _matT(q[:, sl], k[:, sl]) * inv_sqrt_dh
            s_list.append(s_h + (bb[hd, 0] + be_all[l, hd])
                          + binm * ww[hd, 0])
        s_all = jnp.concatenate(s_list, axis=0)
        m1 = jnp.max(s_all, axis=1, keepdims=True)
        t, g = _topk_stats(s_all)
        r = float(TOPK) - g

        tri = (jax.lax.broadcasted_iota(jnp.int32, (CH, CH), 0)
               <= jax.lax.broadcasted_iota(jnp.int32, (CH, CH), 1)
               ).astype(f32)
        p_chunks = []
        off = jnp.zeros((H * N, 1), f32)
        for c in range(N // CH):
            sc = slice(c * CH, (c + 1) * CH)
            s_c = s_all[:, sc]
            eq_c = (s_c == t)
            cum_c = jax.lax.dot_general(
                eq_c.astype(f32), tri, (((1,), (0,)), ((), ())),
                preferred_element_type=f32) + off
            off = cum_c[:, CH - 1:CH]
            sel_c = (s_c > t) | (eq_c & (cum_c <= r))
            p_chunks.append(jnp.where(sel_c, jnp.exp(s_c - m1), 0.0))

        z = p_chunks[0].sum(axis=1, keepdims=True)
        for c in range(1, N // CH):
            z = z + p_chunks[c].sum(axis=1, keepdims=True)

        acc = jnp.zeros((N, D), f32)
        for hd in range(H):
            sl = slice(hd * DH, (hd + 1) * DH)
            rows = slice(hd * N, (hd + 1) * N)
            av = jnp.zeros((N, DH), f32)
            for c in range(N // CH):
                av = av + jax.lax.dot_general(
                    p_chunks[c][rows, :], v[c * CH:(c + 1) * CH, sl],
                    (((1,), (0,)), ((), ())),
                    preferred_element_type=f32)
            acc = acc + _matT(av / z[rows, :], Wo_l[:, sl])

        h_attn = acc + bo_ref[l:l + 1]
        h = _layernorm(residual + h_attn, ln1w_ref[l:l + 1], ln1b_ref[l:l + 1])

        gates = (_matT(h, lstm_Wih_ref[l]) + lstm_bih_ref[l:l + 1]
                 + lstm_bhh_ref[l:l + 1])
        i_g = jax.nn.sigmoid(gates[:, 0:D])
        g_g = jnp.tanh(gates[:, 2 * D:3 * D])
        o_g = jax.nn.sigmoid(gates[:, 3 * D:4 * D])
        h = h + o_g * jnp.tanh(i_g * g_g)

        mu = jnp.mean(h, axis=0, keepdims=True)
        var = jnp.mean((h - mu) ** 2, axis=0, keepdims=True)
        h = gn_g_ref[l:l + 1] * (h - mu) / jnp.sqrt(var + 1e-5) \
            + gn_b_ref[l:l + 1]

        a = jnp.maximum(_matT(h, ffn_W1_ref[l]) + ffn_b1_ref[l:l + 1], 0.0)
        f = _matT(a, ffn_W2_ref[l]) + ffn_b2_ref[l:l + 1]
        h = _layernorm(h + f, ln2w_ref[l:l + 1], ln2b_ref[l:l + 1])
        h = h + h0

    o_ref[...] = _matT(h, out_W_ref[...]) + out_b_ref[...]


def kernel(nodefeature, adjm, params):
    p = params
    args = (
        nodefeature, adjm,
        p['emb_W'], p['emb_b'].reshape(1, D),
        p['edge_W'].T, p['edge_b'].reshape(1, D),
        p['Wq'], p['bq'], p['Wk'], p['bk'], p['Wv'], p['bv'],
        p['We'], p['be'], p['Wo'], p['bo'],
        p['ln1_w'], p['ln1_b'],
        p['lstm_Wih'], p['lstm_bih'], p['lstm_bhh'],
        p['gn_g'], p['gn_b'],
        p['ffn_W1'], p['ffn_b1'], p['ffn_W2'], p['ffn_b2'],
        p['ln2_w'], p['ln2_b'],
        p['out_W'], p['out_b'].reshape(1, D),
    )
    return pl.pallas_call(
        _mynet_body,
        out_shape=jax.ShapeDtypeStruct((N, D), jnp.float32),
    )(*args)


# speedup vs baseline: 18.7781x; 1.1252x over previous
import math

import jax
import jax.numpy as jnp
from jax.experimental import pallas as pl
from jax.experimental.pallas import tpu as pltpu

N = 512
F_IN = 64
H = 8
DH = 16
D = 128
L = 2
TOPK = 16
CH = 128


def _matT(x, w):
    return jax.lax.dot_general(
        x, w, (((x.ndim - 1,), (w.ndim - 1,)), ((), ())),
        preferred_element_type=jnp.float32)


def _layernorm(x, w, b, eps=1e-5):
    mu = jnp.mean(x, axis=-1, keepdims=True)
    var = jnp.mean((x - mu) ** 2, axis=-1, keepdims=True)
    return w * (x - mu) / jnp.sqrt(var + eps) + b


def _topk_stats(s, m1):
    f32 = jnp.float32
    rows = s.shape[0]

    def scoped(cnt_ref, t_ref, g_ref):
        cnt_ref[...] = m1

        def chain(_, carry):
            cnt_ref[...] = jnp.max(
                jnp.where(s < cnt_ref[...], s, -jnp.inf),
                axis=1, keepdims=True)
            return carry

        jax.lax.fori_loop(0, TOPK - 1, chain, 0)
        m = cnt_ref[...]
        c = jnp.sum((s >= m).astype(f32), axis=1, keepdims=True)
        t_ref[...] = m
        g_ref[...] = c - 1.0
        bad = jnp.max(jnp.abs(c - float(TOPK)))

        @pl.when(bad > 0.0)
        def _():
            cnt_ref[...] = jnp.zeros((rows, 1), f32)
            t_ref[...] = jnp.zeros((rows, 1), f32)
            g_ref[...] = jnp.zeros((rows, 1), f32)

            def body(_, w):
                mm = jnp.max(w, axis=1, keepdims=True)
                hit = w == mm
                nh = jnp.sum(hit.astype(f32), axis=1, keepdims=True)
                cnt = cnt_ref[...]
                live = cnt < float(TOPK)
                t_ref[...] = jnp.where(live, mm, t_ref[...])
                g_ref[...] = jnp.where(live, cnt, g_ref[...])
                cnt_ref[...] = cnt + nh
                return jnp.where(hit, -jnp.inf, w)

            jax.lax.fori_loop(0, TOPK, body, s)

        return t_ref[...], g_ref[...], bad

    return pl.run_scoped(scoped,
                         pltpu.VMEM((rows, 1), f32),
                         pltpu.VMEM((rows, 1), f32),
                         pltpu.VMEM((rows, 1), f32))


def _mynet_body(nf_ref, adjm_ref, emb_W_ref, emb_b_ref, edge_w_ref,
                edge_b_ref, Wq_ref, bq_ref, Wk_ref, bk_ref, Wv_ref, bv_ref,
                We_ref, be_ref, Wo_ref, bo_ref, ln1w_ref, ln1b_ref,
                lstm_Wih_ref, lstm_bih_ref, lstm_bhh_ref, gn_g_ref, gn_b_ref,
                ffn_W1_ref, ffn_b1_ref, ffn_W2_ref, ffn_b2_ref, ln2w_ref,
                ln2b_ref, out_W_ref, out_b_ref, o_ref):
    f32 = jnp.float32
    h = _matT(nf_ref[...], emb_W_ref[...]) + emb_b_ref[...]
    h0 = h
    binm = (adjm_ref[...] > 0).astype(f32)
    be_all = be_ref[...]
    inv_sqrt_dh = 1.0 / math.sqrt(DH)

    for l in range(L):
        residual = h
        q = _matT(h, Wq_ref[l]) + bq_ref[l:l + 1]
        k = _matT(h, Wk_ref[l]) + bk_ref[l:l + 1]
        v = _matT(h, Wv_ref[l]) + bv_ref[l:l + 1]
        We_l = We_ref[l]
        ww = _matT(We_l, edge_w_ref[...])
        bb = _matT(We_l, edge_b_ref[...])
        Wo_l = Wo_ref[l]

        s_list = []
        for hd in range(H):
            sl = slice(hd * DH, (hd + 1) * DH)
            s_h = _matT(q[:, sl], k[:, sl]) * inv_sqrt_dh
            s_list.append(s_h + (bb[hd, 0] + be_all[l, hd])
                          + binm * ww[hd, 0])
        s_all = jnp.concatenate(s_list, axis=0)
        m1 = jnp.max(s_all, axis=1, keepdims=True)
        t, g, bad = _topk_stats(s_all, m1)

        def p_scoped(p_ref):
            p_ref[...] = jnp.where(s_all >= t, jnp.exp(s_all - m1), 0.0)

            @pl.when(bad > 0.0)
            def _():
                r = float(TOPK) - g
                tri = (jax.lax.broadcasted_iota(jnp.int32, (CH, CH), 0)
                       <= jax.lax.broadcasted_iota(jnp.int32, (CH, CH), 1)
                       ).astype(f32)
                off = jnp.zeros((H * N, 1), f32)
                for c in range(N // CH):
                    sc = slice(c * CH, (c + 1) * CH)
                    s_c = s_all[:, sc]
                    eq_c = (s_c == t)
                    cum_c = jax.lax.dot_general(
                        eq_c.astype(f32), tri, (((1,), (0,)), ((), ())),
                        preferred_element_type=f32) + off
                    off = cum_c[:, CH - 1:CH]
                    sel_c = (s_c > t) | (eq_c & (cum_c <= r))
                    p_ref[:, sc] = jnp.where(sel_c, jnp.exp(s_c - m1), 0.0)

            return p_ref[...]

        p = pl.run_scoped(p_scoped, pltpu.VMEM((H * N, N), f32))
        z = jnp.sum(p, axis=1, keepdims=True)

        acc = jnp.zeros((N, D), f32)
        for hd in range(H):
            sl = slice(hd * DH, (hd + 1) * DH)
            rows = slice(hd * N, (hd + 1) * N)
            av = jax.lax.dot_general(
                p[rows, :], v[:, sl], (((1,), (0,)), ((), ())),
                preferred_element_type=f32)
            acc = acc + _matT(av / z[rows, :], Wo_l[:, sl])

        h_attn = acc + bo_ref[l:l + 1]
        h = _layernorm(residual + h_attn, ln1w_ref[l:l + 1], ln1b_ref[l:l + 1])

        gates = (_matT(h, lstm_Wih_ref[l]) + lstm_bih_ref[l:l + 1]
                 + lstm_bhh_ref[l:l + 1])
        i_g = jax.nn.sigmoid(gates[:, 0:D])
        g_g = jnp.tanh(gates[:, 2 * D:3 * D])
        o_g = jax.nn.sigmoid(gates[:, 3 * D:4 * D])
        h = h + o_g * jnp.tanh(i_g * g_g)

        mu = jnp.mean(h, axis=0, keepdims=True)
        var = jnp.mean((h - mu) ** 2, axis=0, keepdims=True)
        h = gn_g_ref[l:l + 1] * (h - mu) / jnp.sqrt(var + 1e-5) \
            + gn_b_ref[l:l + 1]

        a = jnp.maximum(_matT(h, ffn_W1_ref[l]) + ffn_b1_ref[l:l + 1], 0.0)
        f = _matT(a, ffn_W2_ref[l]) + ffn_b2_ref[l:l + 1]
        h = _layernorm(h + f, ln2w_ref[l:l + 1], ln2b_ref[l:l + 1])
        h = h + h0

    o_ref[...] = _matT(h, out_W_ref[...]) + out_b_ref[...]


def kernel(nodefeature, adjm, params):
    p = params
    args = (
        nodefeature, adjm,
        p['emb_W'], p['emb_b'].reshape(1, D),
        p['edge_W'].T, p['edge_b'].reshape(1, D),
        p['Wq'], p['bq'], p['Wk'], p['bk'], p['Wv'], p['bv'],
        p['We'], p['be'], p['Wo'], p['bo'],
        p['ln1_w'], p['ln1_b'],
        p['lstm_Wih'], p['lstm_bih'], p['lstm_bhh'],
        p['gn_g'], p['gn_b'],
        p['ffn_W1'], p['ffn_b1'], p['ffn_W2'], p['ffn_b2'],
        p['ln2_w'], p['ln2_b'],
        p['out_W'], p['out_b'].reshape(1, D),
    )
    return pl.pallas_call(
        _mynet_body,
        out_shape=jax.ShapeDtypeStruct((N, D), jnp.float32),
    )(*args)


# speedup vs baseline: 19.1198x; 1.0182x over previous
import math

import jax
import jax.numpy as jnp
from jax.experimental import pallas as pl
from jax.experimental.pallas import tpu as pltpu

N = 512
F_IN = 64
H = 8
DH = 16
D = 128
L = 2
TOPK = 16
CH = 128


def _matT(x, w):
    return jax.lax.dot_general(
        x, w, (((x.ndim - 1,), (w.ndim - 1,)), ((), ())),
        preferred_element_type=jnp.float32)


def _layernorm(x, w, b, eps=1e-5):
    mu = jnp.mean(x, axis=-1, keepdims=True)
    var = jnp.mean((x - mu) ** 2, axis=-1, keepdims=True)
    return w * (x - mu) / jnp.sqrt(var + eps) + b


def _topk_stats(s, m1):
    f32 = jnp.float32
    rows = s.shape[0]

    def scoped(cnt_ref, t_ref, g_ref):
        cnt_ref[...] = m1

        def chain(_, carry):
            cnt_ref[...] = jnp.max(
                jnp.where(s < cnt_ref[...], s, -jnp.inf),
                axis=1, keepdims=True)
            return carry

        jax.lax.fori_loop(0, TOPK - 1, chain, 0)
        m = cnt_ref[...]
        c = jnp.sum((s >= m).astype(f32), axis=1, keepdims=True)
        t_ref[...] = m
        g_ref[...] = c - 1.0
        bad = jnp.max(jnp.abs(c - float(TOPK)))

        @pl.when(bad > 0.0)
        def _():
            cnt_ref[...] = jnp.zeros((rows, 1), f32)
            t_ref[...] = jnp.zeros((rows, 1), f32)
            g_ref[...] = jnp.zeros((rows, 1), f32)

            def body(_, w):
                mm = jnp.max(w, axis=1, keepdims=True)
                hit = w == mm
                nh = jnp.sum(hit.astype(f32), axis=1, keepdims=True)
                cnt = cnt_ref[...]
                live = cnt < float(TOPK)
                t_ref[...] = jnp.where(live, mm, t_ref[...])
                g_ref[...] = jnp.where(live, cnt, g_ref[...])
                cnt_ref[...] = cnt + nh
                return jnp.where(hit, -jnp.inf, w)

            jax.lax.fori_loop(0, TOPK, body, s)

        return t_ref[...], g_ref[...], bad

    return pl.run_scoped(scoped,
                         pltpu.VMEM((rows, 1), f32),
                         pltpu.VMEM((rows, 1), f32),
                         pltpu.VMEM((rows, 1), f32))


def _mynet_body(nf_ref, adjm_ref, emb_W_ref, emb_b_ref, edge_w_ref,
                edge_b_ref, Wq_ref, bq_ref, Wk_ref, bk_ref, Wv_ref, bv_ref,
                We_ref, be_ref, Wo_ref, bo_ref, ln1w_ref, ln1b_ref,
                lstm_Wih_ref, lstm_bih_ref, lstm_bhh_ref, gn_g_ref, gn_b_ref,
                ffn_W1_ref, ffn_b1_ref, ffn_W2_ref, ffn_b2_ref, ln2w_ref,
                ln2b_ref, out_W_ref, out_b_ref, o_ref):
    f32 = jnp.float32
    h = _matT(nf_ref[...], emb_W_ref[...]) + emb_b_ref[...]
    h0 = h
    binm = (adjm_ref[...] > 0).astype(f32)
    be_all = be_ref[...]
    inv_sqrt_dh = 1.0 / math.sqrt(DH)

    for l in range(L):
        residual = h
        q = _matT(h, Wq_ref[l]) + bq_ref[l:l + 1]
        k = _matT(h, Wk_ref[l]) + bk_ref[l:l + 1]
        v = _matT(h, Wv_ref[l]) + bv_ref[l:l + 1]
        We_l = We_ref[l]
        ww = _matT(We_l, edge_w_ref[...])
        bb = _matT(We_l, edge_b_ref[...])
        Wo_l = Wo_ref[l]

        s_list = []
        for hd in range(H):
            sl = slice(hd * DH, (hd + 1) * DH)
            s_h = _matT(q[:, sl], k[:, sl]) * inv_sqrt_dh
            s_list.append(s_h + (bb[hd, 0] + be_all[l, hd])
                          + binm * ww[hd, 0])
        s_all = jnp.concatenate(s_list, axis=0)
        m1 = jnp.max(s_all, axis=1, keepdims=True)
        t, g, bad = _topk_stats(s_all, m1)

        def p_scoped(p_ref):
            p_ref[...] = jnp.where(s_all >= t, jnp.exp(s_all - m1), 0.0)

            @pl.when(bad > 0.0)
            def _():
                r = float(TOPK) - g
                tri = (jax.lax.broadcasted_iota(jnp.int32, (CH, CH), 0)
                       <= jax.lax.broadcasted_iota(jnp.int32, (CH, CH), 1)
                       ).astype(f32)
                off = jnp.zeros((H * N, 1), f32)
                for c in range(N // CH):
                    sc = slice(c * CH, (c + 1) * CH)
                    s_c = s_all[:, sc]
                    eq_c = (s_c == t)
                    cum_c = jax.lax.dot_general(
                        eq_c.astype(f32), tri, (((1,), (0,)), ((), ())),
                        preferred_element_type=f32) + off
                    off = cum_c[:, CH - 1:CH]
                    sel_c = (s_c > t) | (eq_c & (cum_c <= r))
                    p_ref[:, sc] = jnp.where(sel_c, jnp.exp(s_c - m1), 0.0)

            return p_ref[...]

        p = pl.run_scoped(p_scoped, pltpu.VMEM((H * N, N), f32))

        ones_col = jnp.ones((N, 1), f32)
        acc = jnp.zeros((N, D), f32)
        for hd in range(H):
            sl = slice(hd * DH, (hd + 1) * DH)
            rows = slice(hd * N, (hd + 1) * N)
            v_aug = jnp.concatenate([v[:, sl], ones_col], axis=1)
            av = jax.lax.dot_general(
                p[rows, :], v_aug, (((1,), (0,)), ((), ())),
                preferred_element_type=f32)
            acc = acc + _matT(av[:, :DH] / av[:, DH:DH + 1], Wo_l[:, sl])

        h_attn = acc + bo_ref[l:l + 1]
        h = _layernorm(residual + h_attn, ln1w_ref[l:l + 1], ln1b_ref[l:l + 1])

        gates = (_matT(h, lstm_Wih_ref[l]) + lstm_bih_ref[l:l + 1]
                 + lstm_bhh_ref[l:l + 1])
        i_g = jax.nn.sigmoid(gates[:, 0:D])
        g_g = jnp.tanh(gates[:, 2 * D:3 * D])
        o_g = jax.nn.sigmoid(gates[:, 3 * D:4 * D])
        h = h + o_g * jnp.tanh(i_g * g_g)

        mu = jnp.mean(h, axis=0, keepdims=True)
        var = jnp.mean((h - mu) ** 2, axis=0, keepdims=True)
        h = gn_g_ref[l:l + 1] * (h - mu) / jnp.sqrt(var + 1e-5) \
            + gn_b_ref[l:l + 1]

        a = jnp.maximum(_matT(h, ffn_W1_ref[l]) + ffn_b1_ref[l:l + 1], 0.0)
        f = _matT(a, ffn_W2_ref[l]) + ffn_b2_ref[l:l + 1]
        h = _layernorm(h + f, ln2w_ref[l:l + 1], ln2b_ref[l:l + 1])
        h = h + h0

    o_ref[...] = _matT(h, out_W_ref[...]) + out_b_ref[...]


def kernel(nodefeature, adjm, params):
    p = params
    args = (
        nodefeature, adjm,
        p['emb_W'], p['emb_b'].reshape(1, D),
        p['edge_W'].T, p['edge_b'].reshape(1, D),
        p['Wq'], p['bq'], p['Wk'], p['bk'], p['Wv'], p['bv'],
        p['We'], p['be'], p['Wo'], p['bo'],
        p['ln1_w'], p['ln1_b'],
        p['lstm_Wih'], p['lstm_bih'], p['lstm_bhh'],
        p['gn_g'], p['gn_b'],
        p['ffn_W1'], p['ffn_b1'], p['ffn_W2'], p['ffn_b2'],
        p['ln2_w'], p['ln2_b'],
        p['out_W'], p['out_b'].reshape(1, D),
    )
    return pl.pallas_call(
        _mynet_body,
        out_shape=jax.ShapeDtypeStruct((N, D), jnp.float32),
    )(*args)


# speedup vs baseline: 19.4852x; 1.0191x over previous
import math

import jax
import jax.numpy as jnp
from jax.experimental import pallas as pl
from jax.experimental.pallas import tpu as pltpu

N = 512
F_IN = 64
H = 8
DH = 16
D = 128
L = 2
TOPK = 16
CH = 128


def _matT(x, w):
    return jax.lax.dot_general(
        x, w, (((x.ndim - 1,), (w.ndim - 1,)), ((), ())),
        preferred_element_type=jnp.float32)


def _layernorm(x, w, b, eps=1e-5):
    mu = jnp.mean(x, axis=-1, keepdims=True)
    var = jnp.mean((x - mu) ** 2, axis=-1, keepdims=True)
    return w * (x - mu) / jnp.sqrt(var + eps) + b


def _topk_stats(s, m1):
    f32 = jnp.float32
    rows = s.shape[0]

    def scoped(cnt_ref, t_ref, g_ref):
        cnt_ref[...] = m1

        def chain(_, carry):
            cnt_ref[...] = jnp.max(
                jnp.where(s < cnt_ref[...], s, -jnp.inf),
                axis=1, keepdims=True)
            return carry

        jax.lax.fori_loop(0, TOPK - 1, chain, 0)
        m = cnt_ref[...]
        c = jnp.sum((s >= m).astype(f32), axis=1, keepdims=True)
        t_ref[...] = m
        g_ref[...] = c - 1.0
        bad = jnp.max(jnp.abs(c - float(TOPK)))

        @pl.when(bad > 0.0)
        def _():
            cnt_ref[...] = jnp.zeros((rows, 1), f32)
            t_ref[...] = jnp.zeros((rows, 1), f32)
            g_ref[...] = jnp.zeros((rows, 1), f32)

            def body(_, w):
                mm = jnp.max(w, axis=1, keepdims=True)
                hit = w == mm
                nh = jnp.sum(hit.astype(f32), axis=1, keepdims=True)
                cnt = cnt_ref[...]
                live = cnt < float(TOPK)
                t_ref[...] = jnp.where(live, mm, t_ref[...])
                g_ref[...] = jnp.where(live, cnt, g_ref[...])
                cnt_ref[...] = cnt + nh
                return jnp.where(hit, -jnp.inf, w)

            jax.lax.fori_loop(0, TOPK, body, s)

        return t_ref[...], g_ref[...], bad

    return pl.run_scoped(scoped,
                         pltpu.VMEM((rows, 1), f32),
                         pltpu.VMEM((rows, 1), f32),
                         pltpu.VMEM((rows, 1), f32))


def _mynet_body(nf_ref, adjm_ref, emb_W_ref, emb_b_ref, edge_w_ref,
                edge_b_ref, Wq_ref, bq_ref, Wk_ref, bk_ref, Wv_ref, bv_ref,
                We_ref, be_ref, Wo_ref, bo_ref, ln1w_ref, ln1b_ref,
                lstm_Wih_ref, lstm_bih_ref, lstm_bhh_ref, gn_g_ref, gn_b_ref,
                ffn_W1_ref, ffn_b1_ref, ffn_W2_ref, ffn_b2_ref, ln2w_ref,
                ln2b_ref, out_W_ref, out_b_ref, o_ref):
    f32 = jnp.float32
    h = _matT(nf_ref[...], emb_W_ref[...]) + emb_b_ref[...]
    h0 = h
    binm = (adjm_ref[...] > 0).astype(f32)
    be_all = be_ref[...]
    inv_sqrt_dh = 1.0 / math.sqrt(DH)

    for l in range(L):
        residual = h
        q = _matT(h, Wq_ref[l]) + bq_ref[l:l + 1]
        k = _matT(h, Wk_ref[l]) + bk_ref[l:l + 1]
        v = _matT(h, Wv_ref[l]) + bv_ref[l:l + 1]
        We_l = We_ref[l]
        ww = _matT(We_l, edge_w_ref[...])
        bb = _matT(We_l, edge_b_ref[...])
        Wo_l = Wo_ref[l]

        def attn_scoped(s_ref, p_ref):
            for hd in range(H):
                sl = slice(hd * DH, (hd + 1) * DH)
                s_h = _matT(q[:, sl], k[:, sl]) * inv_sqrt_dh
                s_ref[hd * N:(hd + 1) * N, :] = (
                    s_h + (bb[hd, 0] + be_all[l, hd]) + binm * ww[hd, 0])
            s_all = s_ref[...]
            m1 = jnp.max(s_all, axis=1, keepdims=True)
            t, g, bad = _topk_stats(s_all, m1)
            p_ref[...] = jnp.where(s_all >= t, jnp.exp(s_all - m1), 0.0)

            @pl.when(bad > 0.0)
            def _():
                r = float(TOPK) - g
                tri = (jax.lax.broadcasted_iota(jnp.int32, (CH, CH), 0)
                       <= jax.lax.broadcasted_iota(jnp.int32, (CH, CH), 1)
                       ).astype(f32)
                off = jnp.zeros((H * N, 1), f32)
                for c in range(N // CH):
                    sc = slice(c * CH, (c + 1) * CH)
                    s_c = s_all[:, sc]
                    eq_c = (s_c == t)
                    cum_c = jax.lax.dot_general(
                        eq_c.astype(f32), tri, (((1,), (0,)), ((), ())),
                        preferred_element_type=f32) + off
                    off = cum_c[:, CH - 1:CH]
                    sel_c = (s_c > t) | (eq_c & (cum_c <= r))
                    p_ref[:, sc] = jnp.where(sel_c, jnp.exp(s_c - m1), 0.0)

            ones_col = jnp.ones((N, 1), f32)
            acc = jnp.zeros((N, D), f32)
            for hd in range(H):
                sl = slice(hd * DH, (hd + 1) * DH)
                rows = slice(hd * N, (hd + 1) * N)
                v_aug = jnp.concatenate([v[:, sl], ones_col], axis=1)
                av = jax.lax.dot_general(
                    p_ref[rows, :], v_aug, (((1,), (0,)), ((), ())),
                    preferred_element_type=f32)
                acc = acc + _matT(av[:, :DH] / av[:, DH:DH + 1], Wo_l[:, sl])
            return acc

        acc = pl.run_scoped(attn_scoped,
                            pltpu.VMEM((H * N, N), f32),
                            pltpu.VMEM((H * N, N), f32))
        h_attn = acc + bo_ref[l:l + 1]
        h = _layernorm(residual + h_attn, ln1w_ref[l:l + 1], ln1b_ref[l:l + 1])

        gates = (_matT(h, lstm_Wih_ref[l]) + lstm_bih_ref[l:l + 1]
                 + lstm_bhh_ref[l:l + 1])
        i_g = jax.nn.sigmoid(gates[:, 0:D])
        g_g = jnp.tanh(gates[:, 2 * D:3 * D])
        o_g = jax.nn.sigmoid(gates[:, 3 * D:4 * D])
        h = h + o_g * jnp.tanh(i_g * g_g)

        mu = jnp.mean(h, axis=0, keepdims=True)
        var = jnp.mean((h - mu) ** 2, axis=0, keepdims=True)
        h = gn_g_ref[l:l + 1] * (h - mu) / jnp.sqrt(var + 1e-5) \
            + gn_b_ref[l:l + 1]

        a = jnp.maximum(_matT(h, ffn_W1_ref[l]) + ffn_b1_ref[l:l + 1], 0.0)
        f = _matT(a, ffn_W2_ref[l]) + ffn_b2_ref[l:l + 1]
        h = _layernorm(h + f, ln2w_ref[l:l + 1], ln2b_ref[l:l + 1])
        h = h + h0

    o_ref[...] = _matT(h, out_W_ref[...]) + out_b_ref[...]


def kernel(nodefeature, adjm, params):
    p = params
    args = (
        nodefeature, adjm,
        p['emb_W'], p['emb_b'].reshape(1, D),
        p['edge_W'].T, p['edge_b'].reshape(1, D),
        p['Wq'], p['bq'], p['Wk'], p['bk'], p['Wv'], p['bv'],
        p['We'], p['be'], p['Wo'], p['bo'],
        p['ln1_w'], p['ln1_b'],
        p['lstm_Wih'], p['lstm_bih'], p['lstm_bhh'],
        p['gn_g'], p['gn_b'],
        p['ffn_W1'], p['ffn_b1'], p['ffn_W2'], p['ffn_b2'],
        p['ln2_w'], p['ln2_b'],
        p['out_W'], p['out_b'].reshape(1, D),
    )
    return pl.pallas_call(
        _mynet_body,
        out_shape=jax.ShapeDtypeStruct((N, D), jnp.float32),
    )(*args)
